# baseline (speedup 1.0000x reference)
import functools

import jax
import jax.numpy as jnp
from jax import lax
from jax.experimental import pallas as pl
from jax.experimental.pallas import tpu as pltpu
from jax.experimental.pallas import tpu_sc as plsc

EMBED_DIM = 64
NUM_WORKERS = 32
B_PER_CHUNK = 2


def _sc_gather(indices, table):
  batch, hist = indices.shape
  b_per_w = batch // NUM_WORKERS
  n_chunks = b_per_w // B_PER_CHUNK
  n_pairs = n_chunks // 2

  mesh = plsc.VectorSubcoreMesh(core_axis_name="c", subcore_axis_name="s")

  @functools.partial(
      pl.kernel,
      mesh=mesh,
      compiler_params=pltpu.CompilerParams(use_tc_tiling_on_sc=False),
      out_type=jax.ShapeDtypeStruct((batch, hist, EMBED_DIM), jnp.float32),
      scratch_types=[
          pltpu.VMEM((B_PER_CHUNK, hist), jnp.int32),
          pltpu.VMEM((B_PER_CHUNK, hist), jnp.int32),
          pltpu.VMEM((B_PER_CHUNK, hist, EMBED_DIM), jnp.float32),
          pltpu.VMEM((B_PER_CHUNK, hist, EMBED_DIM), jnp.float32),
          pltpu.SemaphoreType.DMA,
          pltpu.SemaphoreType.DMA,
      ],
  )
  def k(idx_hbm, table_hbm, out_hbm, idx0, idx1, rows0, rows1, sem0, sem1):
    wid = lax.axis_index("s") * 2 + lax.axis_index("c")
    b_base = wid * b_per_w

    def gather_chunk(idx_v, rows_v, sem, c):
      b0 = b_base + c * B_PER_CHUNK
      pltpu.sync_copy(idx_hbm.at[pl.ds(b0, B_PER_CHUNK)], idx_v)
      for j in range(B_PER_CHUNK):
        pltpu.async_copy(table_hbm.at[idx_v.at[j]], rows_v.at[j], sem)

    def wait_and_store(rows_v, sem, c):
      b0 = b_base + c * B_PER_CHUNK
      pltpu.make_async_copy(out_hbm.at[pl.ds(0, B_PER_CHUNK)], rows_v,
                            sem).wait()
      pltpu.sync_copy(rows_v, out_hbm.at[pl.ds(b0, B_PER_CHUNK)])

    def body(p, carry):
      ca = 2 * p

      gather_chunk(idx0, rows0, sem0, ca)

      @pl.when(p > 0)
      def _():
        wait_and_store(rows1, sem1, ca - 1)

      gather_chunk(idx1, rows1, sem1, ca + 1)
      wait_and_store(rows0, sem0, ca)
      return carry

    lax.fori_loop(0, n_pairs, body, 0)
    wait_and_store(rows1, sem1, n_chunks - 1)

  return k(indices, table)


def kernel(indices, glove_vectors):
  return _sc_gather(indices.astype(jnp.int32), glove_vectors)


# speedup vs baseline: 1.1223x; 1.1223x over previous
import functools

import jax
import jax.numpy as jnp
from jax import lax
from jax.experimental import pallas as pl
from jax.experimental.pallas import tpu as pltpu
from jax.experimental.pallas import tpu_sc as plsc

EMBED_DIM = 64
NUM_WORKERS = 32
B_PER_CHUNK = 2


def _sc_gather(indices, table):
  batch, hist = indices.shape
  b_per_w = batch // NUM_WORKERS
  n_chunks = b_per_w // B_PER_CHUNK
  n_pairs = n_chunks // 2

  mesh = plsc.VectorSubcoreMesh(core_axis_name="c", subcore_axis_name="s")

  @functools.partial(
      pl.kernel,
      mesh=mesh,
      compiler_params=pltpu.CompilerParams(use_tc_tiling_on_sc=False),
      out_type=jax.ShapeDtypeStruct((batch, hist, EMBED_DIM), jnp.float32),
      scratch_types=[
          pltpu.VMEM((B_PER_CHUNK, hist), jnp.int32),
          pltpu.VMEM((B_PER_CHUNK, hist), jnp.int32),
          pltpu.VMEM((B_PER_CHUNK, hist, EMBED_DIM), jnp.float32),
          pltpu.VMEM((B_PER_CHUNK, hist, EMBED_DIM), jnp.float32),
          pltpu.SemaphoreType.DMA,
          pltpu.SemaphoreType.DMA,
      ],
  )
  def k(idx_hbm, table_hbm, out_hbm, idx0, idx1, rows0, rows1, sem0, sem1):
    wid = lax.axis_index("s") * 2 + lax.axis_index("c")
    b_base = wid * b_per_w

    def gather_chunk(idx_v, rows_v, sem, c):
      b0 = b_base + c * B_PER_CHUNK
      pltpu.sync_copy(idx_hbm.at[pl.ds(b0, B_PER_CHUNK)], idx_v)
      for j in range(B_PER_CHUNK):
        pltpu.async_copy(table_hbm.at[idx_v.at[j]], rows_v.at[j], sem)

    def wait_and_store(rows_v, sem, c):
      b0 = b_base + c * B_PER_CHUNK
      pltpu.make_async_copy(out_hbm.at[pl.ds(0, B_PER_CHUNK)], rows_v,
                            sem).wait()
      pltpu.sync_copy(rows_v, out_hbm.at[pl.ds(b0, B_PER_CHUNK)])

    def body(p, carry):
      ca = 2 * p

      gather_chunk(idx0, rows0, sem0, ca)

      @pl.when(p > 0)
      def _():
        wait_and_store(rows1, sem1, ca - 1)

      gather_chunk(idx1, rows1, sem1, ca + 1)
      wait_and_store(rows0, sem0, ca)
      return carry

    lax.fori_loop(0, n_pairs, body, 0)
    wait_and_store(rows1, sem1, n_chunks - 1)

  return k(indices, table)


_TC_BLK = 2048


def _tc_compact(table_t):
  d, v = table_t.shape
  grid = (v + _TC_BLK - 1) // _TC_BLK
  half = _TC_BLK // 2

  def body(in_ref, out_ref):
    xt = in_ref[...].T
    out_ref[...] = jnp.concatenate([xt[:half], xt[half:]], axis=1)

  return pl.pallas_call(
      body,
      grid=(grid,),
      in_specs=[pl.BlockSpec((d, _TC_BLK), lambda i: (0, i))],
      out_specs=pl.BlockSpec((half, 2 * d), lambda i: (i, 0)),
      out_shape=jax.ShapeDtypeStruct((grid * half, 2 * d), jnp.float32),
  )(table_t)


def kernel(indices, glove_vectors):
  v, d = glove_vectors.shape
  packed = _tc_compact(glove_vectors.T)
  table_lin = packed.reshape(2 * packed.shape[0], d)
  idx = indices.astype(jnp.int32)
  idx = (idx & ~2047) | ((idx & 1023) << 1) | ((idx >> 10) & 1)
  return _sc_gather(idx, table_lin)


# speedup vs baseline: 1.3487x; 1.2017x over previous
import functools

import jax
import jax.numpy as jnp
from jax import lax
from jax.experimental import pallas as pl
from jax.experimental.pallas import tpu as pltpu
from jax.experimental.pallas import tpu_sc as plsc

EMBED_DIM = 64
NUM_WORKERS = 32
B_PER_CHUNK = 2


def _sc_gather(indices, table):
  batch, hist = indices.shape
  b_per_w = batch // NUM_WORKERS
  n_chunks = b_per_w // B_PER_CHUNK
  n_pairs = n_chunks // 2

  mesh = plsc.VectorSubcoreMesh(core_axis_name="c", subcore_axis_name="s")

  @functools.partial(
      pl.kernel,
      mesh=mesh,
      compiler_params=pltpu.CompilerParams(use_tc_tiling_on_sc=False),
      out_type=jax.ShapeDtypeStruct((batch, hist, EMBED_DIM), jnp.float32),
      scratch_types=[
          pltpu.VMEM((B_PER_CHUNK, hist), jnp.int32),
          pltpu.VMEM((B_PER_CHUNK, hist), jnp.int32),
          pltpu.VMEM((B_PER_CHUNK, hist, EMBED_DIM), jnp.float32),
          pltpu.VMEM((B_PER_CHUNK, hist, EMBED_DIM), jnp.float32),
          pltpu.SemaphoreType.DMA,
          pltpu.SemaphoreType.DMA,
      ],
  )
  def k(idx_hbm, table_hbm, out_hbm, idx0, idx1, rows0, rows1, sem0, sem1):
    wid = lax.axis_index("s") * 2 + lax.axis_index("c")
    b_base = wid * b_per_w

    def gather_chunk(idx_v, rows_v, sem, c):
      b0 = b_base + c * B_PER_CHUNK
      pltpu.sync_copy(idx_hbm.at[pl.ds(b0, B_PER_CHUNK)], idx_v)
      for j in range(B_PER_CHUNK):
        pltpu.async_copy(table_hbm.at[idx_v.at[j]], rows_v.at[j], sem)

    def wait_and_store(rows_v, sem, c):
      b0 = b_base + c * B_PER_CHUNK
      pltpu.make_async_copy(out_hbm.at[pl.ds(0, B_PER_CHUNK)], rows_v,
                            sem).wait()
      pltpu.sync_copy(rows_v, out_hbm.at[pl.ds(b0, B_PER_CHUNK)])

    def body(p, carry):
      ca = 2 * p

      gather_chunk(idx0, rows0, sem0, ca)

      @pl.when(p > 0)
      def _():
        wait_and_store(rows1, sem1, ca - 1)

      gather_chunk(idx1, rows1, sem1, ca + 1)
      wait_and_store(rows0, sem0, ca)
      return carry

    lax.fori_loop(0, n_pairs, body, 0)
    wait_and_store(rows1, sem1, n_chunks - 1)

  return k(indices, table)


_TC_BLK = 8192


def _tc_compact(table_t):
  d, v = table_t.shape
  grid = (v + _TC_BLK - 1) // _TC_BLK
  half = _TC_BLK // 2

  def body(in_ref, out_ref):
    xt = in_ref[...].T
    out_ref[...] = jnp.concatenate([xt[:half], xt[half:]], axis=1)

  return pl.pallas_call(
      body,
      grid=(grid,),
      in_specs=[pl.BlockSpec((d, _TC_BLK), lambda i: (0, i))],
      out_specs=pl.BlockSpec((half, 2 * d), lambda i: (i, 0)),
      out_shape=jax.ShapeDtypeStruct((grid * half, 2 * d), jnp.float32),
  )(table_t)


def kernel(indices, glove_vectors):
  v, d = glove_vectors.shape
  packed = _tc_compact(glove_vectors.T)
  table_lin = packed.reshape(2 * packed.shape[0], d)
  half = _TC_BLK // 2
  shift = half.bit_length() - 1
  idx = indices.astype(jnp.int32)
  idx = (idx & ~(_TC_BLK - 1)) | ((idx & (half - 1)) << 1) | ((idx >> shift) & 1)
  return _sc_gather(idx, table_lin)


# speedup vs baseline: 1.4216x; 1.0541x over previous
import functools

import jax
import jax.numpy as jnp
from jax import lax
from jax.experimental import pallas as pl
from jax.experimental.pallas import tpu as pltpu
from jax.experimental.pallas import tpu_sc as plsc

EMBED_DIM = 64
NUM_WORKERS = 32
B_PER_CHUNK = 2


def _sc_gather(indices, table):
  batch, hist = indices.shape
  b_per_w = batch // NUM_WORKERS
  n_chunks = b_per_w // B_PER_CHUNK
  n_pairs = n_chunks // 2

  mesh = plsc.VectorSubcoreMesh(core_axis_name="c", subcore_axis_name="s")

  @functools.partial(
      pl.kernel,
      mesh=mesh,
      compiler_params=pltpu.CompilerParams(use_tc_tiling_on_sc=False),
      out_type=jax.ShapeDtypeStruct((batch, hist, EMBED_DIM), jnp.float32),
      scratch_types=[
          pltpu.VMEM((B_PER_CHUNK, hist), jnp.int32),
          pltpu.VMEM((B_PER_CHUNK, hist), jnp.int32),
          pltpu.VMEM((B_PER_CHUNK, hist, EMBED_DIM), jnp.float32),
          pltpu.VMEM((B_PER_CHUNK, hist, EMBED_DIM), jnp.float32),
          pltpu.SemaphoreType.DMA,
          pltpu.SemaphoreType.DMA,
      ],
  )
  def k(idx_hbm, table_hbm, out_hbm, idx0, idx1, rows0, rows1, sem0, sem1):
    wid = lax.axis_index("s") * 2 + lax.axis_index("c")
    b_base = wid * b_per_w

    def gather_chunk(idx_v, rows_v, sem, c):
      b0 = b_base + c * B_PER_CHUNK
      pltpu.sync_copy(idx_hbm.at[pl.ds(b0, B_PER_CHUNK)], idx_v)
      for j in range(B_PER_CHUNK):
        pltpu.async_copy(table_hbm.at[idx_v.at[j]], rows_v.at[j], sem)

    def wait_and_store(rows_v, sem, c):
      b0 = b_base + c * B_PER_CHUNK
      pltpu.make_async_copy(out_hbm.at[pl.ds(0, B_PER_CHUNK)], rows_v,
                            sem).wait()
      pltpu.sync_copy(rows_v, out_hbm.at[pl.ds(b0, B_PER_CHUNK)])

    def body(p, carry):
      ca = 2 * p

      gather_chunk(idx0, rows0, sem0, ca)

      @pl.when(p > 0)
      def _():
        wait_and_store(rows1, sem1, ca - 1)

      gather_chunk(idx1, rows1, sem1, ca + 1)
      wait_and_store(rows0, sem0, ca)
      return carry

    lax.fori_loop(0, n_pairs, body, 0)
    wait_and_store(rows1, sem1, n_chunks - 1)

  return k(indices, table)


_TC_BLK = 32768


def _tc_compact(table_t):
  d, v = table_t.shape
  grid = (v + _TC_BLK - 1) // _TC_BLK
  half = _TC_BLK // 2

  def body(in_ref, out_ref):
    xt = in_ref[...].T
    out_ref[...] = jnp.concatenate([xt[:half], xt[half:]], axis=1)

  return pl.pallas_call(
      body,
      grid=(grid,),
      in_specs=[pl.BlockSpec((d, _TC_BLK), lambda i: (0, i))],
      out_specs=pl.BlockSpec((half, 2 * d), lambda i: (i, 0)),
      out_shape=jax.ShapeDtypeStruct((grid * half, 2 * d), jnp.float32),
  )(table_t)


def kernel(indices, glove_vectors):
  v, d = glove_vectors.shape
  packed = _tc_compact(glove_vectors.T)
  table_lin = packed.reshape(2 * packed.shape[0], d)
  half = _TC_BLK // 2
  shift = half.bit_length() - 1
  idx = indices.astype(jnp.int32)
  idx = (idx & ~(_TC_BLK - 1)) | ((idx & (half - 1)) << 1) | ((idx >> shift) & 1)
  return _sc_gather(idx, table_lin)


# speedup vs baseline: 2.1822x; 1.5351x over previous
import functools

import jax
import jax.numpy as jnp
from jax import lax
from jax.experimental import pallas as pl
from jax.experimental.pallas import tpu as pltpu
from jax.experimental.pallas import tpu_sc as plsc

EMBED_DIM = 64
NUM_WORKERS = 32
B_PER_CHUNK = 2


def _sc_gather(indices, table):
  batch, hist = indices.shape
  b_per_w = batch // NUM_WORKERS
  n_chunks = b_per_w // B_PER_CHUNK
  n_pairs = n_chunks // 2

  mesh = plsc.VectorSubcoreMesh(core_axis_name="c", subcore_axis_name="s")

  @functools.partial(
      pl.kernel,
      mesh=mesh,
      compiler_params=pltpu.CompilerParams(use_tc_tiling_on_sc=False),
      out_type=jax.ShapeDtypeStruct((batch, hist, 2 * EMBED_DIM), jnp.float32),
      scratch_types=[
          pltpu.VMEM((B_PER_CHUNK, hist), jnp.int32),
          pltpu.VMEM((B_PER_CHUNK, hist), jnp.int32),
          pltpu.VMEM((B_PER_CHUNK, hist, EMBED_DIM), jnp.float32),
          pltpu.VMEM((B_PER_CHUNK, hist, EMBED_DIM), jnp.float32),
          pltpu.SemaphoreType.DMA,
          pltpu.SemaphoreType.DMA,
      ],
  )
  def k(idx_hbm, table_hbm, out_hbm, idx0, idx1, rows0, rows1, sem0, sem1):
    wid = lax.axis_index("s") * 2 + lax.axis_index("c")
    b_base = wid * b_per_w

    def gather_chunk(idx_v, rows_v, sem, c):
      b0 = b_base + c * B_PER_CHUNK
      pltpu.sync_copy(idx_hbm.at[pl.ds(b0, B_PER_CHUNK)], idx_v)
      for j in range(B_PER_CHUNK):
        pltpu.async_copy(table_hbm.at[idx_v.at[j]], rows_v.at[j], sem)

    def wait_and_store(rows_v, sem, c):
      b0 = b_base + c * B_PER_CHUNK
      pltpu.make_async_copy(
          out_hbm.at[pl.ds(0, B_PER_CHUNK), :, pl.ds(0, EMBED_DIM)],
          rows_v, sem).wait()
      pltpu.sync_copy(
          rows_v,
          out_hbm.at[pl.ds(b0, B_PER_CHUNK), :, pl.ds(0, EMBED_DIM)])

    def body(p, carry):
      ca = 2 * p

      gather_chunk(idx0, rows0, sem0, ca)

      @pl.when(p > 0)
      def _():
        wait_and_store(rows1, sem1, ca - 1)

      gather_chunk(idx1, rows1, sem1, ca + 1)
      wait_and_store(rows0, sem0, ca)
      return carry

    lax.fori_loop(0, n_pairs, body, 0)
    wait_and_store(rows1, sem1, n_chunks - 1)

  return k(indices, table)


_TC_BLK = 32768


def _tc_compact(table_t):
  d, v = table_t.shape
  grid = (v + _TC_BLK - 1) // _TC_BLK
  half = _TC_BLK // 2

  def body(in_ref, out_ref):
    xt = in_ref[...].T
    out_ref[...] = jnp.concatenate([xt[:half], xt[half:]], axis=1)

  return pl.pallas_call(
      body,
      grid=(grid,),
      in_specs=[pl.BlockSpec((d, _TC_BLK), lambda i: (0, i))],
      out_specs=pl.BlockSpec((half, 2 * d), lambda i: (i, 0)),
      out_shape=jax.ShapeDtypeStruct((grid * half, 2 * d), jnp.float32),
  )(table_t)


def kernel(indices, glove_vectors):
  v, d = glove_vectors.shape
  packed = _tc_compact(glove_vectors.T)
  table_lin = packed.reshape(2 * packed.shape[0], d)
  half = _TC_BLK // 2
  shift = half.bit_length() - 1
  idx = indices.astype(jnp.int32)
  idx = (idx & ~(_TC_BLK - 1)) | ((idx & (half - 1)) << 1) | ((idx >> shift) & 1)
  return _sc_gather(idx, table_lin)[:, :, :EMBED_DIM]
